# baseline (speedup 1.0000x reference)
import jax
import jax.numpy as jnp
from jax import lax
from jax.experimental import pallas as pl
from jax.experimental.pallas import tpu as pltpu
from jax.experimental.pallas import tpu_sc as plsc

_BS = 16
_ATOM = 10000
_NEI = 65
_KN = 16
_CHUNK = 200
_SPLITS = (2400, 2400, 2400, 2800)
_STRIDE = _ATOM + 4


def _rsqrt(x):
    i = plsc.bitcast(x, jnp.int32)
    y = plsc.bitcast(jnp.int32(0x5F3759DF) - (i >> 1), jnp.float32)
    y = y * (1.5 - 0.5 * x * y * y)
    return y


def _make_conv(gbase, half):
    nchunk = half // _CHUNK
    natoms = 2 * half

    def _body(idx_hbm, pos_hbm, out_hbm, pos_v, idx0, idx1, out0,
              isem0, isem1, osem0):
        b = lax.axis_index("s")
        h = lax.axis_index("c")

        pltpu.sync_copy(pos_hbm.at[b], pos_v)

        compa = (jnp.arange(_KN, dtype=jnp.int32) % 3) * _STRIDE

        def compute(idx_v, out_v, astart):
            @plsc.parallel_loop(0, _CHUNK, unroll=8)
            def _(i):
                cvec = plsc.load_gather(
                    pos_v, [jnp.full((_KN,), gbase + astart + i) + compa]
                )
                cx = jnp.full((_KN,), cvec[0])
                cy = jnp.full((_KN,), cvec[1])
                cz = jnp.full((_KN,), cvec[2])
                row0 = idx_v[i, pl.ds(0, 16)]
                nvec = plsc.load_gather(
                    pos_v, [jnp.full((_KN,), row0[0]) + compa]
                )
                d0x = jnp.full((_KN,), nvec[0]) - cx
                d0y = jnp.full((_KN,), nvec[1]) - cy
                d0z = jnp.full((_KN,), nvec[2]) - cz
                s0 = d0x * d0x + d0y * d0y + d0z * d0z
                r0 = _rsqrt(s0)

                acc = jnp.zeros((_KN,), jnp.float32)
                for g in range(4):
                    idxg = idx_v[i, pl.ds(1 + 16 * g, 16)]
                    gx = plsc.load_gather(pos_v, [idxg])
                    gy = plsc.load_gather(pos_v, [idxg + _STRIDE])
                    gz = plsc.load_gather(pos_v, [idxg + 2 * _STRIDE])
                    dx = gx - cx
                    dy = gy - cy
                    dz = gz - cz
                    q = dx * d0x + dy * d0y + dz * d0z
                    ss = dx * dx + dy * dy + dz * dz
                    r = _rsqrt(ss)
                    acc = acc + jnp.maximum(q, 0.0) * r
                out_v[i] = acc * r0

        def aoff(c):
            return pl.multiple_of(h * half + c * _CHUNK, 8)

        def istart(c, buf, sem):
            pltpu.make_async_copy(
                idx_hbm.at[b, pl.ds(aoff(c), _CHUNK)], buf, sem
            ).start()

        def iwait(buf, sem):
            pltpu.make_async_copy(
                idx_hbm.at[b, pl.ds(aoff(0), _CHUNK)], buf, sem
            ).wait()

        def ostart(c, buf, sem):
            pltpu.make_async_copy(
                buf, out_hbm.at[b, pl.ds(aoff(c), _CHUNK)], sem
            ).start()

        def owait(buf, sem):
            pltpu.make_async_copy(
                buf, out_hbm.at[b, pl.ds(aoff(0), _CHUNK)], sem
            ).wait()

        istart(0, idx0, isem0)

        def chunk_loop(c, _):
            @pl.when(c + 1 < nchunk)
            def _():
                @pl.when((c + 1) % 2 == 0)
                def _():
                    istart(c + 1, idx0, isem0)

                @pl.when((c + 1) % 2 == 1)
                def _():
                    istart(c + 1, idx1, isem1)

            @pl.when(c >= 1)
            def _():
                owait(out0, osem0)

            @pl.when(c % 2 == 0)
            def _():
                iwait(idx0, isem0)
                compute(idx0, out0, aoff(c))

            @pl.when(c % 2 == 1)
            def _():
                iwait(idx1, isem1)
                compute(idx1, out0, aoff(c))

            ostart(c, out0, osem0)
            return 0

        lax.fori_loop(0, nchunk, chunk_loop, 0)
        owait(out0, osem0)

    mesh = plsc.VectorSubcoreMesh(core_axis_name="c", subcore_axis_name="s")
    return pl.kernel(
        _body,
        out_type=jax.ShapeDtypeStruct((_BS, natoms, _KN), jnp.float32),
        mesh=mesh,
        scratch_types=[
            pltpu.VMEM((3 * _STRIDE,), jnp.float32),
            pltpu.VMEM((_CHUNK, _NEI), jnp.int32),
            pltpu.VMEM((_CHUNK, _NEI), jnp.int32),
            pltpu.VMEM((_CHUNK, _KN), jnp.float32),
            pltpu.SemaphoreType.DMA,
            pltpu.SemaphoreType.DMA,
            pltpu.SemaphoreType.DMA,
        ],
        compiler_params=pltpu.CompilerParams(needs_layout_passes=False),
    )


@jax.jit
def _atom_conv(neighbor_index, pos_soa):
    parts = []
    gbase = 0
    for n in _SPLITS:
        parts.append(
            _make_conv(gbase, n // 2)(
                neighbor_index[:, gbase:gbase + n], pos_soa
            )
        )
        gbase += n
    return jnp.concatenate(parts, axis=1)


def kernel(neighbor_index, atoms, angle_weights):
    del angle_weights
    pos_soa = jnp.pad(
        atoms.transpose(0, 2, 1), ((0, 0), (0, 0), (0, _STRIDE - _ATOM))
    ).reshape(_BS, 3 * _STRIDE)
    return _atom_conv(neighbor_index, pos_soa)


# speedup vs baseline: 1.0008x; 1.0008x over previous
import jax
import jax.numpy as jnp
from jax import lax
from jax.experimental import pallas as pl
from jax.experimental.pallas import tpu as pltpu
from jax.experimental.pallas import tpu_sc as plsc

_BS = 16
_ATOM = 10000
_NEI = 65
_KN = 16
_CHUNK = 200
_SPLITS = (3600, 6400)
_STRIDE = _ATOM + 4


def _rsqrt(x):
    i = plsc.bitcast(x, jnp.int32)
    y = plsc.bitcast(jnp.int32(0x5F3759DF) - (i >> 1), jnp.float32)
    y = y * (1.5 - 0.5 * x * y * y)
    return y


def _make_conv(gbase, half):
    nchunk = half // _CHUNK
    natoms = 2 * half

    def _body(idx_hbm, pos_hbm, out_hbm, pos_v, idx0, idx1, out0,
              isem0, isem1, osem0):
        b = lax.axis_index("s")
        h = lax.axis_index("c")

        pltpu.sync_copy(pos_hbm.at[b], pos_v)

        compa = (jnp.arange(_KN, dtype=jnp.int32) % 3) * _STRIDE

        def compute(idx_v, out_v, astart):
            @plsc.parallel_loop(0, _CHUNK, unroll=8)
            def _(i):
                cvec = plsc.load_gather(
                    pos_v, [jnp.full((_KN,), gbase + astart + i) + compa]
                )
                cx = jnp.full((_KN,), cvec[0])
                cy = jnp.full((_KN,), cvec[1])
                cz = jnp.full((_KN,), cvec[2])
                row0 = idx_v[i, pl.ds(0, 16)]
                nvec = plsc.load_gather(
                    pos_v, [jnp.full((_KN,), row0[0]) + compa]
                )
                d0x = jnp.full((_KN,), nvec[0]) - cx
                d0y = jnp.full((_KN,), nvec[1]) - cy
                d0z = jnp.full((_KN,), nvec[2]) - cz
                s0 = d0x * d0x + d0y * d0y + d0z * d0z
                r0 = _rsqrt(s0)

                acc = jnp.zeros((_KN,), jnp.float32)
                for g in range(4):
                    idxg = idx_v[i, pl.ds(1 + 16 * g, 16)]
                    gx = plsc.load_gather(pos_v, [idxg])
                    gy = plsc.load_gather(pos_v, [idxg + _STRIDE])
                    gz = plsc.load_gather(pos_v, [idxg + 2 * _STRIDE])
                    dx = gx - cx
                    dy = gy - cy
                    dz = gz - cz
                    q = dx * d0x + dy * d0y + dz * d0z
                    ss = dx * dx + dy * dy + dz * dz
                    r = _rsqrt(ss)
                    acc = acc + jnp.maximum(q, 0.0) * r
                out_v[i] = acc * r0

        def aoff(c):
            return pl.multiple_of(h * half + c * _CHUNK, 8)

        def istart(c, buf, sem):
            pltpu.make_async_copy(
                idx_hbm.at[b, pl.ds(aoff(c), _CHUNK)], buf, sem
            ).start()

        def iwait(buf, sem):
            pltpu.make_async_copy(
                idx_hbm.at[b, pl.ds(aoff(0), _CHUNK)], buf, sem
            ).wait()

        def ostart(c, buf, sem):
            pltpu.make_async_copy(
                buf, out_hbm.at[b, pl.ds(aoff(c), _CHUNK)], sem
            ).start()

        def owait(buf, sem):
            pltpu.make_async_copy(
                buf, out_hbm.at[b, pl.ds(aoff(0), _CHUNK)], sem
            ).wait()

        istart(0, idx0, isem0)

        def chunk_loop(c, _):
            @pl.when(c + 1 < nchunk)
            def _():
                @pl.when((c + 1) % 2 == 0)
                def _():
                    istart(c + 1, idx0, isem0)

                @pl.when((c + 1) % 2 == 1)
                def _():
                    istart(c + 1, idx1, isem1)

            @pl.when(c >= 1)
            def _():
                owait(out0, osem0)

            @pl.when(c % 2 == 0)
            def _():
                iwait(idx0, isem0)
                compute(idx0, out0, aoff(c))

            @pl.when(c % 2 == 1)
            def _():
                iwait(idx1, isem1)
                compute(idx1, out0, aoff(c))

            ostart(c, out0, osem0)
            return 0

        lax.fori_loop(0, nchunk, chunk_loop, 0)
        owait(out0, osem0)

    mesh = plsc.VectorSubcoreMesh(core_axis_name="c", subcore_axis_name="s")
    return pl.kernel(
        _body,
        out_type=jax.ShapeDtypeStruct((_BS, natoms, _KN), jnp.float32),
        mesh=mesh,
        scratch_types=[
            pltpu.VMEM((3 * _STRIDE,), jnp.float32),
            pltpu.VMEM((_CHUNK, _NEI), jnp.int32),
            pltpu.VMEM((_CHUNK, _NEI), jnp.int32),
            pltpu.VMEM((_CHUNK, _KN), jnp.float32),
            pltpu.SemaphoreType.DMA,
            pltpu.SemaphoreType.DMA,
            pltpu.SemaphoreType.DMA,
        ],
        compiler_params=pltpu.CompilerParams(needs_layout_passes=False),
    )


@jax.jit
def _atom_conv(neighbor_index, pos_soa):
    parts = []
    gbase = 0
    for n in _SPLITS:
        parts.append(
            _make_conv(gbase, n // 2)(
                neighbor_index[:, gbase:gbase + n], pos_soa
            )
        )
        gbase += n
    return jnp.concatenate(parts, axis=1)


def kernel(neighbor_index, atoms, angle_weights):
    del angle_weights
    pos_soa = jnp.pad(
        atoms.transpose(0, 2, 1), ((0, 0), (0, 0), (0, _STRIDE - _ATOM))
    ).reshape(_BS, 3 * _STRIDE)
    return _atom_conv(neighbor_index, pos_soa)


# speedup vs baseline: 1.0230x; 1.0222x over previous
import jax
import jax.numpy as jnp
from jax import lax
from jax.experimental import pallas as pl
from jax.experimental.pallas import tpu as pltpu
from jax.experimental.pallas import tpu_sc as plsc

_BS = 16
_ATOM = 10000
_NEI = 65
_KN = 16
_CHUNK = 200
_SPLITS = (5200, 4800)
_STRIDE = _ATOM + 4


def _rsqrt(x):
    i = plsc.bitcast(x, jnp.int32)
    y = plsc.bitcast(jnp.int32(0x5F3759DF) - (i >> 1), jnp.float32)
    y = y * (1.5 - 0.5 * x * y * y)
    return y


def _make_conv(gbase, half):
    nchunk = half // _CHUNK
    natoms = 2 * half

    def _body(idx_hbm, pos_hbm, out_hbm, pos_v, idx0, idx1, out0,
              isem0, isem1, osem0):
        b = lax.axis_index("s")
        h = lax.axis_index("c")

        pltpu.sync_copy(pos_hbm.at[b], pos_v)

        compa = (jnp.arange(_KN, dtype=jnp.int32) % 3) * _STRIDE

        def compute(idx_v, out_v, astart):
            @plsc.parallel_loop(0, _CHUNK, unroll=8)
            def _(i):
                cvec = plsc.load_gather(
                    pos_v, [jnp.full((_KN,), gbase + astart + i) + compa]
                )
                cx = jnp.full((_KN,), cvec[0])
                cy = jnp.full((_KN,), cvec[1])
                cz = jnp.full((_KN,), cvec[2])
                row0 = idx_v[i, pl.ds(0, 16)]
                nvec = plsc.load_gather(
                    pos_v, [jnp.full((_KN,), row0[0]) + compa]
                )
                d0x = jnp.full((_KN,), nvec[0]) - cx
                d0y = jnp.full((_KN,), nvec[1]) - cy
                d0z = jnp.full((_KN,), nvec[2]) - cz
                s0 = d0x * d0x + d0y * d0y + d0z * d0z
                r0 = _rsqrt(s0)

                acc = jnp.zeros((_KN,), jnp.float32)
                for g in range(4):
                    idxg = idx_v[i, pl.ds(1 + 16 * g, 16)]
                    gx = plsc.load_gather(pos_v, [idxg])
                    gy = plsc.load_gather(pos_v, [idxg + _STRIDE])
                    gz = plsc.load_gather(pos_v, [idxg + 2 * _STRIDE])
                    dx = gx - cx
                    dy = gy - cy
                    dz = gz - cz
                    q = dx * d0x + dy * d0y + dz * d0z
                    ss = dx * dx + dy * dy + dz * dz
                    r = _rsqrt(ss)
                    acc = acc + jnp.maximum(q, 0.0) * r
                out_v[i] = acc * r0

        def aoff(c):
            return pl.multiple_of(h * half + c * _CHUNK, 8)

        def istart(c, buf, sem):
            pltpu.make_async_copy(
                idx_hbm.at[b, pl.ds(aoff(c), _CHUNK)], buf, sem
            ).start()

        def iwait(buf, sem):
            pltpu.make_async_copy(
                idx_hbm.at[b, pl.ds(aoff(0), _CHUNK)], buf, sem
            ).wait()

        def ostart(c, buf, sem):
            pltpu.make_async_copy(
                buf, out_hbm.at[b, pl.ds(aoff(c), _CHUNK)], sem
            ).start()

        def owait(buf, sem):
            pltpu.make_async_copy(
                buf, out_hbm.at[b, pl.ds(aoff(0), _CHUNK)], sem
            ).wait()

        istart(0, idx0, isem0)

        def chunk_loop(c, _):
            @pl.when(c + 1 < nchunk)
            def _():
                @pl.when((c + 1) % 2 == 0)
                def _():
                    istart(c + 1, idx0, isem0)

                @pl.when((c + 1) % 2 == 1)
                def _():
                    istart(c + 1, idx1, isem1)

            @pl.when(c >= 1)
            def _():
                owait(out0, osem0)

            @pl.when(c % 2 == 0)
            def _():
                iwait(idx0, isem0)
                compute(idx0, out0, aoff(c))

            @pl.when(c % 2 == 1)
            def _():
                iwait(idx1, isem1)
                compute(idx1, out0, aoff(c))

            ostart(c, out0, osem0)
            return 0

        lax.fori_loop(0, nchunk, chunk_loop, 0)
        owait(out0, osem0)

    mesh = plsc.VectorSubcoreMesh(core_axis_name="c", subcore_axis_name="s")
    return pl.kernel(
        _body,
        out_type=jax.ShapeDtypeStruct((_BS, natoms, _KN), jnp.float32),
        mesh=mesh,
        scratch_types=[
            pltpu.VMEM((3 * _STRIDE,), jnp.float32),
            pltpu.VMEM((_CHUNK, _NEI), jnp.int32),
            pltpu.VMEM((_CHUNK, _NEI), jnp.int32),
            pltpu.VMEM((_CHUNK, _KN), jnp.float32),
            pltpu.SemaphoreType.DMA,
            pltpu.SemaphoreType.DMA,
            pltpu.SemaphoreType.DMA,
        ],
        compiler_params=pltpu.CompilerParams(needs_layout_passes=False),
    )


@jax.jit
def _atom_conv(neighbor_index, pos_soa):
    parts = []
    gbase = 0
    for n in _SPLITS:
        parts.append(
            _make_conv(gbase, n // 2)(
                neighbor_index[:, gbase:gbase + n], pos_soa
            )
        )
        gbase += n
    return jnp.concatenate(parts, axis=1)


def kernel(neighbor_index, atoms, angle_weights):
    del angle_weights
    pos_soa = jnp.pad(
        atoms.transpose(0, 2, 1), ((0, 0), (0, 0), (0, _STRIDE - _ATOM))
    ).reshape(_BS, 3 * _STRIDE)
    return _atom_conv(neighbor_index, pos_soa)
